# baseline (speedup 1.0000x reference)
import functools

import jax
import jax.numpy as jnp
from jax import lax
from jax.experimental import pallas as pl
from jax.experimental.pallas import tpu as pltpu
from jax.experimental.pallas import tpu_sc as plsc

VOCAB = 100000
HIDDEN = 128
SEQ = 200
EPS = 1e-12
NC, NS = 2, 16
NW = NC * NS
NSEQ = 1024
CH = 40
NCH = SEQ // CH
TCB = 32
NCHUNK = 2
CNSEQ = NSEQ // NCHUNK
CBLK = CNSEQ // TCB


def _make_gather(nseq):
    seq_per_w = nseq // NW
    npair = seq_per_w // 2

    @functools.partial(
        pl.kernel,
        out_type=jax.ShapeDtypeStruct((nseq * SEQ, HIDDEN), jnp.float32),
        mesh=plsc.VectorSubcoreMesh(
            core_axis_name="c", subcore_axis_name="s",
            num_cores=NC, num_subcores=NS,
        ),
        scratch_types=[
            pltpu.VMEM((NCH, CH), jnp.int32),
            pltpu.VMEM((NCH, CH), jnp.int32),
            pltpu.VMEM((2, SEQ, HIDDEN), jnp.float32),
            pltpu.SemaphoreType.DMA,
            pltpu.SemaphoreType.DMA,
            pltpu.SemaphoreType.DMA,
            pltpu.SemaphoreType.DMA,
            pltpu.SemaphoreType.DMA,
            pltpu.SemaphoreType.DMA,
        ],
    )
    def gather_kernel(ids_hbm, wword_hbm, out_hbm, idx0_v, idx1_v, rows_v,
                      sem_g0, sem_g1, sem_i0, sem_i1, sem_o0, sem_o1):
        wid = lax.axis_index("s") * NC + lax.axis_index("c")
        wbase = wid * seq_per_w * SEQ

        sem_g = (sem_g0, sem_g1)
        sem_i = (sem_i0, sem_i1)
        sem_o = (sem_o0, sem_o1)
        rows = (rows_v.at[0], rows_v.at[1])
        idxb = (idx0_v, idx1_v)

        def issue_gather(b):
            for c in range(NCH):
                pltpu.async_copy(
                    wword_hbm.at[idxb[b].at[c]],
                    rows[b].at[pl.ds(c * CH, CH)],
                    sem_g[b],
                )

        def wait_gather(b):
            pltpu.make_async_copy(
                out_hbm.at[pl.ds(0, SEQ)], rows[b], sem_g[b]
            ).wait()

        def issue_idx(b, tokbase):
            for c in range(NCH):
                pltpu.async_copy(
                    ids_hbm.at[pl.ds(tokbase + c * CH, CH)],
                    idxb[b].at[c],
                    sem_i[b],
                )

        def wait_idx(b):
            for c in range(NCH):
                pltpu.make_async_copy(
                    ids_hbm.at[pl.ds(0, CH)], idxb[b].at[c], sem_i[b]
                ).wait()

        def issue_out(b, tokbase):
            pltpu.async_copy(rows[b], out_hbm.at[pl.ds(tokbase, SEQ)], sem_o[b])

        def wait_out(b):
            pltpu.make_async_copy(
                rows[b], out_hbm.at[pl.ds(0, SEQ)], sem_o[b]
            ).wait()

        for c in range(NCH):
            pltpu.sync_copy(
                ids_hbm.at[pl.ds(wbase + c * CH, CH)], idxb[0].at[c]
            )
        issue_gather(0)
        issue_idx(1, wbase + SEQ)

        def pair_body(g, carry):
            pbase = wbase + 2 * g * SEQ

            wait_gather(0)
            wait_idx(1)

            @pl.when(g > 0)
            def _():
                wait_out(1)

            issue_gather(1)

            @pl.when(g < npair - 1)
            def _():
                issue_idx(0, pbase + 2 * SEQ)

            issue_out(0, pbase)

            wait_gather(1)
            wait_out(0)

            @pl.when(g < npair - 1)
            def _():
                wait_idx(0)
                issue_gather(0)
                issue_idx(1, pbase + 3 * SEQ)

            issue_out(1, pbase + SEQ)
            return carry

        lax.fori_loop(0, npair, pair_body, 0)
        wait_out(1)

    return gather_kernel


_gather_chunk = _make_gather(CNSEQ)


def _ln_math(x_ref, tt_ref, pos_ref, t0_ref, t1_ref, gamma_ref, beta_ref,
             o_ref):
    x = x_ref[...].reshape(TCB, SEQ, HIDDEN)
    ttf = tt_ref[0].astype(jnp.float32)[:, :, None]
    tsel = t0_ref[...][None, None, :] + ttf * (
        t1_ref[...] - t0_ref[...]
    )[None, None, :]
    x = x + pos_ref[...][None, :, :] + tsel
    mean = jnp.mean(x, axis=-1, keepdims=True)
    var = jnp.mean(jnp.square(x - mean), axis=-1, keepdims=True)
    xhat = (x - mean) * lax.rsqrt(var + EPS)
    y = xhat * gamma_ref[...][None, None, :] + beta_ref[...][None, None, :]
    o_ref[...] = y.reshape(TCB * SEQ, HIDDEN)


def _make_ln(chunk, aliased):
    def body(*refs):
        if aliased:
            _ln_math(*refs[:7], refs[8])
        else:
            _ln_math(*refs)

    in_specs = [
        pl.BlockSpec((TCB * SEQ, HIDDEN), lambda i: (i, 0)),
        pl.BlockSpec((1, TCB, SEQ), lambda i: (0, chunk * CBLK + i, 0)),
        pl.BlockSpec((SEQ, HIDDEN), lambda i: (0, 0)),
        pl.BlockSpec((HIDDEN,), lambda i: (0,)),
        pl.BlockSpec((HIDDEN,), lambda i: (0,)),
        pl.BlockSpec((HIDDEN,), lambda i: (0,)),
        pl.BlockSpec((HIDDEN,), lambda i: (0,)),
    ]
    kwargs = {}
    if aliased:
        in_specs.append(pl.BlockSpec(memory_space=pl.ANY))
        kwargs["input_output_aliases"] = {7: 0}
    return pl.pallas_call(
        body,
        out_shape=jax.ShapeDtypeStruct((NSEQ * SEQ, HIDDEN), jnp.float32),
        grid=(CBLK,),
        in_specs=in_specs,
        out_specs=pl.BlockSpec(
            (TCB * SEQ, HIDDEN), lambda i: (chunk * CBLK + i, 0)
        ),
        **kwargs,
    )


_ln_first = _make_ln(0, aliased=False)
_ln_chain = [_make_ln(c, aliased=True) for c in range(1, NCHUNK)]


def kernel(input_ids, token_type_ids, W_word, W_pos, W_type, gamma, beta):
    b, s = input_ids.shape
    ids = input_ids.reshape(-1).astype(jnp.int32)
    tt3 = token_type_ids.astype(jnp.int32).reshape(1, NSEQ, SEQ)
    pos = W_pos[:SEQ]
    t0, t1 = W_type[0], W_type[1]

    words = [
        _gather_chunk(
            lax.slice_in_dim(ids, c * CNSEQ * SEQ, (c + 1) * CNSEQ * SEQ),
            W_word,
        )
        for c in range(NCHUNK)
    ]
    out = _ln_first(words[0], tt3, pos, t0, t1, gamma, beta)
    for c in range(1, NCHUNK):
        out = _ln_chain[c - 1](words[c], tt3, pos, t0, t1, gamma, beta, out)
    return out.reshape(b, s, HIDDEN)


# speedup vs baseline: 1.0154x; 1.0154x over previous
import functools

import jax
import jax.numpy as jnp
from jax import lax
from jax.experimental import pallas as pl
from jax.experimental.pallas import tpu as pltpu
from jax.experimental.pallas import tpu_sc as plsc

VOCAB = 100000
HIDDEN = 128
SEQ = 200
EPS = 1e-12
NC, NS = 2, 16
NW = NC * NS
NSEQ = 1024
CH = 40
NCH = SEQ // CH
TCB = 32
NCHUNK = 4
CNSEQ = NSEQ // NCHUNK
CBLK = CNSEQ // TCB


def _make_gather(nseq):
    seq_per_w = nseq // NW
    npair = seq_per_w // 2

    @functools.partial(
        pl.kernel,
        out_type=jax.ShapeDtypeStruct((nseq * SEQ, HIDDEN), jnp.float32),
        mesh=plsc.VectorSubcoreMesh(
            core_axis_name="c", subcore_axis_name="s",
            num_cores=NC, num_subcores=NS,
        ),
        scratch_types=[
            pltpu.VMEM((NCH, CH), jnp.int32),
            pltpu.VMEM((NCH, CH), jnp.int32),
            pltpu.VMEM((2, SEQ, HIDDEN), jnp.float32),
            pltpu.SemaphoreType.DMA,
            pltpu.SemaphoreType.DMA,
            pltpu.SemaphoreType.DMA,
            pltpu.SemaphoreType.DMA,
            pltpu.SemaphoreType.DMA,
            pltpu.SemaphoreType.DMA,
        ],
    )
    def gather_kernel(ids_hbm, wword_hbm, out_hbm, idx0_v, idx1_v, rows_v,
                      sem_g0, sem_g1, sem_i0, sem_i1, sem_o0, sem_o1):
        wid = lax.axis_index("s") * NC + lax.axis_index("c")
        wbase = wid * seq_per_w * SEQ

        sem_g = (sem_g0, sem_g1)
        sem_i = (sem_i0, sem_i1)
        sem_o = (sem_o0, sem_o1)
        rows = (rows_v.at[0], rows_v.at[1])
        idxb = (idx0_v, idx1_v)

        def issue_gather(b):
            for c in range(NCH):
                pltpu.async_copy(
                    wword_hbm.at[idxb[b].at[c]],
                    rows[b].at[pl.ds(c * CH, CH)],
                    sem_g[b],
                )

        def wait_gather(b):
            pltpu.make_async_copy(
                out_hbm.at[pl.ds(0, SEQ)], rows[b], sem_g[b]
            ).wait()

        def issue_idx(b, tokbase):
            for c in range(NCH):
                pltpu.async_copy(
                    ids_hbm.at[pl.ds(tokbase + c * CH, CH)],
                    idxb[b].at[c],
                    sem_i[b],
                )

        def wait_idx(b):
            for c in range(NCH):
                pltpu.make_async_copy(
                    ids_hbm.at[pl.ds(0, CH)], idxb[b].at[c], sem_i[b]
                ).wait()

        def issue_out(b, tokbase):
            pltpu.async_copy(rows[b], out_hbm.at[pl.ds(tokbase, SEQ)], sem_o[b])

        def wait_out(b):
            pltpu.make_async_copy(
                rows[b], out_hbm.at[pl.ds(0, SEQ)], sem_o[b]
            ).wait()

        for c in range(NCH):
            pltpu.sync_copy(
                ids_hbm.at[pl.ds(wbase + c * CH, CH)], idxb[0].at[c]
            )
        issue_gather(0)
        issue_idx(1, wbase + SEQ)

        def pair_body(g, carry):
            pbase = wbase + 2 * g * SEQ

            wait_gather(0)
            wait_idx(1)

            @pl.when(g > 0)
            def _():
                wait_out(1)

            issue_gather(1)

            @pl.when(g < npair - 1)
            def _():
                issue_idx(0, pbase + 2 * SEQ)

            issue_out(0, pbase)

            wait_gather(1)
            wait_out(0)

            @pl.when(g < npair - 1)
            def _():
                wait_idx(0)
                issue_gather(0)
                issue_idx(1, pbase + 3 * SEQ)

            issue_out(1, pbase + SEQ)
            return carry

        lax.fori_loop(0, npair, pair_body, 0)
        wait_out(1)

    return gather_kernel


_gather_chunk = _make_gather(CNSEQ)


def _ln_math(x_ref, tt_ref, pos_ref, t0_ref, t1_ref, gamma_ref, beta_ref,
             o_ref):
    x = x_ref[...].reshape(TCB, SEQ, HIDDEN)
    ttf = tt_ref[0].astype(jnp.float32)[:, :, None]
    tsel = t0_ref[...][None, None, :] + ttf * (
        t1_ref[...] - t0_ref[...]
    )[None, None, :]
    x = x + pos_ref[...][None, :, :] + tsel
    mean = jnp.mean(x, axis=-1, keepdims=True)
    var = jnp.mean(jnp.square(x - mean), axis=-1, keepdims=True)
    xhat = (x - mean) * lax.rsqrt(var + EPS)
    y = xhat * gamma_ref[...][None, None, :] + beta_ref[...][None, None, :]
    o_ref[...] = y.reshape(TCB * SEQ, HIDDEN)


def _make_ln(chunk, aliased):
    def body(*refs):
        if aliased:
            _ln_math(*refs[:7], refs[8])
        else:
            _ln_math(*refs)

    in_specs = [
        pl.BlockSpec((TCB * SEQ, HIDDEN), lambda i: (i, 0)),
        pl.BlockSpec((1, TCB, SEQ), lambda i: (0, chunk * CBLK + i, 0)),
        pl.BlockSpec((SEQ, HIDDEN), lambda i: (0, 0)),
        pl.BlockSpec((HIDDEN,), lambda i: (0,)),
        pl.BlockSpec((HIDDEN,), lambda i: (0,)),
        pl.BlockSpec((HIDDEN,), lambda i: (0,)),
        pl.BlockSpec((HIDDEN,), lambda i: (0,)),
    ]
    kwargs = {}
    if aliased:
        in_specs.append(pl.BlockSpec(memory_space=pl.ANY))
        kwargs["input_output_aliases"] = {7: 0}
    return pl.pallas_call(
        body,
        out_shape=jax.ShapeDtypeStruct((NSEQ * SEQ, HIDDEN), jnp.float32),
        grid=(CBLK,),
        in_specs=in_specs,
        out_specs=pl.BlockSpec(
            (TCB * SEQ, HIDDEN), lambda i: (chunk * CBLK + i, 0)
        ),
        **kwargs,
    )


_ln_first = _make_ln(0, aliased=False)
_ln_chain = [_make_ln(c, aliased=True) for c in range(1, NCHUNK)]


def kernel(input_ids, token_type_ids, W_word, W_pos, W_type, gamma, beta):
    b, s = input_ids.shape
    ids = input_ids.reshape(-1).astype(jnp.int32)
    tt3 = token_type_ids.astype(jnp.int32).reshape(1, NSEQ, SEQ)
    pos = W_pos[:SEQ]
    t0, t1 = W_type[0], W_type[1]

    words = [
        _gather_chunk(
            lax.slice_in_dim(ids, c * CNSEQ * SEQ, (c + 1) * CNSEQ * SEQ),
            W_word,
        )
        for c in range(NCHUNK)
    ]
    out = _ln_first(words[0], tt3, pos, t0, t1, gamma, beta)
    for c in range(1, NCHUNK):
        out = _ln_chain[c - 1](words[c], tt3, pos, t0, t1, gamma, beta, out)
    return out.reshape(b, s, HIDDEN)
